# Initial kernel scaffold; baseline (speedup 1.0000x reference)
import jax
import jax.numpy as jnp
from jax.experimental import pallas as pl


def kernel(hidden_states, router_kernel, router_bias, gate_up_proj, gate_up_proj_bias, down_proj, down_proj_bias):
    raise NotImplementedError("write your pallas kernel here")


# speedup vs baseline: 2.3614x; 2.3614x over previous
import functools

import jax
import jax.numpy as jnp
from jax.experimental import pallas as pl

B, S, H = 1, 2048, 1024
E, K, M = 8, 2, 2048
T = B * S
ALPHA = 1.702
LIMIT = 7.0

MT = 512
NMT = M // MT


def _router_body(x_ref, wr_ref, br_ref, scores_ref):
    x = x_ref[...]
    logits = jnp.dot(x, wr_ref[...], preferred_element_type=jnp.float32)
    logits = logits + br_ref[...][None, :]
    iota = jax.lax.broadcasted_iota(jnp.int32, (T, E), 1)
    neg_inf = jnp.float32(-jnp.inf)

    m1 = jnp.max(logits, axis=1, keepdims=True)
    i1 = jnp.min(jnp.where(logits == m1, iota, E), axis=1, keepdims=True)
    masked = jnp.where(iota == i1, neg_inf, logits)
    m2 = jnp.max(masked, axis=1, keepdims=True)
    i2 = jnp.min(jnp.where(masked == m2, iota, E), axis=1, keepdims=True)

    e2 = jnp.exp(m2 - m1)
    denom = 1.0 + e2
    w1 = 1.0 / denom
    w2 = e2 / denom
    scores_ref[...] = jnp.where(iota == i1, w1, 0.0) + jnp.where(iota == i2, w2, 0.0)


def _expert_body(x_ref, wg_ref, wu_ref, w2_ref, bg_ref, bu_ref, bd_ref,
                 s_ref, out_ref):
    e = pl.program_id(0)
    m = pl.program_id(1)

    x = x_ref[...]
    gate = jnp.dot(x, wg_ref[0], preferred_element_type=jnp.float32)
    gate = gate + bg_ref[0]
    up = jnp.dot(x, wu_ref[0], preferred_element_type=jnp.float32)
    up = up + bu_ref[0]

    gate = jnp.clip(gate, -1e9, LIMIT)
    up = jnp.clip(up, -LIMIT, LIMIT)
    glu = gate * jax.nn.sigmoid(gate * ALPHA)
    act = (up + 1.0) * glu

    onehot = (jax.lax.broadcasted_iota(jnp.int32, (E, 1), 0) == e
              ).astype(jnp.float32)
    s_col = jnp.dot(s_ref[...], onehot, preferred_element_type=jnp.float32)

    partial = jnp.dot(act * s_col, w2_ref[0], preferred_element_type=jnp.float32)

    @pl.when(m == 0)
    def _():
        partial_b = partial + s_col * bd_ref[0]

        @pl.when(e == 0)
        def _():
            out_ref[...] = partial_b

        @pl.when(e != 0)
        def _():
            out_ref[...] += partial_b

    @pl.when(m != 0)
    def _():
        out_ref[...] += partial


@jax.jit
def kernel(hidden_states, router_kernel, router_bias, gate_up_proj,
           gate_up_proj_bias, down_proj, down_proj_bias):
    flat = hidden_states.reshape(T, H)

    scores = pl.pallas_call(
        _router_body,
        out_shape=jax.ShapeDtypeStruct((T, E), jnp.float32),
    )(flat, router_kernel, router_bias)

    gu = gate_up_proj.reshape(E, H, M, 2)
    wg = gu[..., 0]
    wu = gu[..., 1]
    gub = gate_up_proj_bias.reshape(E, M, 2)
    bg = gub[..., 0].reshape(E, 1, M)
    bu = gub[..., 1].reshape(E, 1, M)
    bd = down_proj_bias.reshape(E, 1, H)

    out = pl.pallas_call(
        _expert_body,
        grid=(E, NMT),
        in_specs=[
            pl.BlockSpec((T, H), lambda e, m: (0, 0)),
            pl.BlockSpec((1, H, MT), lambda e, m: (e, 0, m)),
            pl.BlockSpec((1, H, MT), lambda e, m: (e, 0, m)),
            pl.BlockSpec((1, MT, H), lambda e, m: (e, m, 0)),
            pl.BlockSpec((1, 1, MT), lambda e, m: (e, 0, m)),
            pl.BlockSpec((1, 1, MT), lambda e, m: (e, 0, m)),
            pl.BlockSpec((1, 1, H), lambda e, m: (e, 0, 0)),
            pl.BlockSpec((T, E), lambda e, m: (0, 0)),
        ],
        out_specs=pl.BlockSpec((T, H), lambda e, m: (0, 0)),
        out_shape=jax.ShapeDtypeStruct((T, H), jnp.float32),
    )(flat, wg, wu, down_proj, bg, bu, bd, scores)

    return out.reshape(B, S, H), scores
